# baseline (speedup 1.0000x reference)
import functools

import jax
import jax.numpy as jnp
from jax.experimental import pallas as pl
from jax.experimental.pallas import tpu as pltpu

_B, _S, _D = 2, 4096, 1024
_E = 8
_K = 2
_DFF = 4096

_T = _B * _S
_T2 = _T * _K
_BM = 512
_NT = _T2 // _BM + _E - 1
_RP = _NT * _BM
_BK = 2048
_NK = _DFF // _BK


def _serp(t, k):
    return jax.lax.select((t % 2) == 0, k, _NK - 1 - k)


def _ffn_body(bid_ref, eid_ref, x_ref, w1_ref, b1_ref, w2_ref, b2_ref, out_ref):
    k = pl.program_id(1)
    x = x_ref[...].astype(jnp.bfloat16)
    w1 = w1_ref[0].astype(jnp.bfloat16)
    h = jnp.dot(x, w1, preferred_element_type=jnp.float32) + b1_ref[0]
    h = jax.nn.gelu(h).astype(jnp.bfloat16)
    part = jnp.dot(h, w2_ref[0].astype(jnp.bfloat16),
                   preferred_element_type=jnp.float32)

    @pl.when(k == 0)
    def _():
        out_ref[...] = part + b2_ref[0]

    @pl.when(k != 0)
    def _():
        out_ref[...] = out_ref[...] + part


def _grouped_ffn(sorted_x, bid, eid, W1, b1, W2, b2):
    grid_spec = pltpu.PrefetchScalarGridSpec(
        num_scalar_prefetch=2,
        grid=(_NT, _NK),
        in_specs=[
            pl.BlockSpec((_BM, _D), lambda t, k, bid, eid: (bid[t], 0)),
            pl.BlockSpec((1, _D, _BK), lambda t, k, bid, eid: (eid[t], 0, _serp(t, k))),
            pl.BlockSpec((1, 1, _BK), lambda t, k, bid, eid: (eid[t], 0, _serp(t, k))),
            pl.BlockSpec((1, _BK, _D), lambda t, k, bid, eid: (eid[t], _serp(t, k), 0)),
            pl.BlockSpec((1, 1, _D), lambda t, k, bid, eid: (eid[t], 0, 0)),
        ],
        out_specs=pl.BlockSpec((_BM, _D), lambda t, k, bid, eid: (bid[t], 0)),
    )
    return pl.pallas_call(
        _ffn_body,
        grid_spec=grid_spec,
        out_shape=jax.ShapeDtypeStruct((_RP, _D), jnp.float32),
        compiler_params=pltpu.CompilerParams(
            dimension_semantics=("arbitrary", "arbitrary"),
        ),
    )(bid, eid, sorted_x, W1, b1.reshape(_E, 1, _DFF), W2, b2.reshape(_E, 1, _D))


def kernel(x, gate_W, gate_b, W1, b1, W2, b2):
    x_flat = x.reshape(_T, _D)

    logits = x_flat @ gate_W + gate_b
    top_vals, top_idx = jax.lax.top_k(logits, _K)
    w = jax.nn.softmax(top_vals.astype(jnp.float32), axis=1)

    flat_e = top_idx.reshape(-1).astype(jnp.int32)
    oh = (flat_e[:, None] == jnp.arange(_E, dtype=jnp.int32)[None, :])
    cum = jnp.cumsum(oh.astype(jnp.int32), axis=0)
    g = cum[-1]
    rank = jnp.take_along_axis(cum, flat_e[:, None], axis=1)[:, 0] - 1

    nb_e = (g + _BM - 1) // _BM
    csnb = jnp.cumsum(nb_e)
    nbp = csnb[-1]
    off_pad = jnp.concatenate([jnp.zeros((1,), jnp.int32),
                               (csnb[:-1] * _BM).astype(jnp.int32)])
    pos = off_pad[flat_e] + rank

    t_ar = jnp.arange(_NT, dtype=jnp.int32)
    bid = jnp.minimum(t_ar, nbp - 1).astype(jnp.int32)
    eid = jnp.searchsorted(csnb, bid, side="right").astype(jnp.int32)

    slot_tok = jnp.zeros((_RP,), jnp.int32).at[pos].set(
        jnp.arange(_T2, dtype=jnp.int32) // _K)
    sorted_x = x_flat[slot_tok]

    contrib = _grouped_ffn(sorted_x, bid, eid, W1, b1, W2, b2)

    pos2 = pos.reshape(_T, _K)
    out = (contrib[pos2[:, 0]] * w[:, 0:1] + contrib[pos2[:, 1]] * w[:, 1:2])
    return out.reshape(_B, _S, _D)


# speedup vs baseline: 1.1685x; 1.1685x over previous
import jax
import jax.numpy as jnp
from jax import lax
from jax.experimental import pallas as pl
from jax.experimental.pallas import tpu as pltpu
from jax.experimental.pallas import tpu_sc as plsc

_B, _S, _D = 2, 4096, 1024
_E = 8
_K = 2
_DFF = 4096

_T = _B * _S
_T2 = _T * _K
_BM = 512
_NT = _T2 // _BM + _E - 1
_RP = _NT * _BM
_BK = 2048
_NK = _DFF // _BK

_NC = 2
_NS = 16
_NW = _NC * _NS
_TPW = _T // _NW
_DCH = 64
_CCH = 32

_MESH = plsc.VectorSubcoreMesh(core_axis_name="c", subcore_axis_name="s")


def _dispatch_body(x_hbm, p0_hbm, p1_hbm, out_hbm, rows_v, i0_v, i1_v,
                   sem0, sem1):
    wid = lax.axis_index("s") * _NC + lax.axis_index("c")
    base = wid * _TPW

    def chunk(c, carry):
        b = base + c * _DCH
        pltpu.sync_copy(x_hbm.at[pl.ds(b, _DCH)], rows_v)
        pltpu.sync_copy(p0_hbm.at[pl.ds(b, _DCH)], i0_v)
        pltpu.sync_copy(p1_hbm.at[pl.ds(b, _DCH)], i1_v)
        c0 = pltpu.async_copy(rows_v, out_hbm.at[i0_v], sem0)
        c1 = pltpu.async_copy(rows_v, out_hbm.at[i1_v], sem1)
        c0.wait()
        c1.wait()
        return carry

    lax.fori_loop(0, _TPW // _DCH, chunk, 0)


def _sc_dispatch(x_flat, pos0, pos1):
    return pl.kernel(
        _dispatch_body,
        out_type=jax.ShapeDtypeStruct((_RP, _D), jnp.float32),
        mesh=_MESH,
        scratch_types=[
            pltpu.VMEM((_DCH, _D), jnp.float32),
            pltpu.VMEM((_DCH,), jnp.int32),
            pltpu.VMEM((_DCH,), jnp.int32),
            pltpu.SemaphoreType.DMA,
            pltpu.SemaphoreType.DMA,
        ],
    )(x_flat, pos0, pos1)


def _combine_body(ctr_hbm, p0_hbm, p1_hbm, w0_hbm, w1_hbm, y_hbm,
                  buf0, buf1, outv, i0_v, i1_v, w0_v, w1_v, sem0, sem1):
    wid = lax.axis_index("s") * _NC + lax.axis_index("c")
    base = wid * _TPW

    def chunk(c, carry):
        b = base + c * _CCH
        pltpu.sync_copy(p0_hbm.at[pl.ds(b, _CCH)], i0_v)
        pltpu.sync_copy(p1_hbm.at[pl.ds(b, _CCH)], i1_v)
        pltpu.sync_copy(w0_hbm.at[pl.ds(b, _CCH)], w0_v)
        pltpu.sync_copy(w1_hbm.at[pl.ds(b, _CCH)], w1_v)
        g0 = pltpu.async_copy(ctr_hbm.at[i0_v], buf0, sem0)
        g1 = pltpu.async_copy(ctr_hbm.at[i1_v], buf1, sem1)
        g0.wait()
        g1.wait()

        def tok(i, cc):
            w0r = w0_v[i]
            w1r = w1_v[i]
            for j in range(_D // 16):
                sl = pl.ds(j * 16, 16)
                outv[i, sl] = buf0[i, sl] * w0r + buf1[i, sl] * w1r
            return cc

        lax.fori_loop(0, _CCH, tok, 0)
        pltpu.sync_copy(outv, y_hbm.at[pl.ds(b, _CCH)])
        return carry

    lax.fori_loop(0, _TPW // _CCH, chunk, 0)


def _sc_combine(contrib, pos0, pos1, w0x, w1x):
    return pl.kernel(
        _combine_body,
        out_type=jax.ShapeDtypeStruct((_T, _D), jnp.float32),
        mesh=_MESH,
        scratch_types=[
            pltpu.VMEM((_CCH, _D), jnp.float32),
            pltpu.VMEM((_CCH, _D), jnp.float32),
            pltpu.VMEM((_CCH, _D), jnp.float32),
            pltpu.VMEM((_CCH,), jnp.int32),
            pltpu.VMEM((_CCH,), jnp.int32),
            pltpu.VMEM((_CCH, 16), jnp.float32),
            pltpu.VMEM((_CCH, 16), jnp.float32),
            pltpu.SemaphoreType.DMA,
            pltpu.SemaphoreType.DMA,
        ],
    )(contrib, pos0, pos1, w0x, w1x)


def _serp(t, k):
    return jax.lax.select((t % 2) == 0, k, _NK - 1 - k)


def _ffn_body(bid_ref, eid_ref, x_ref, w1_ref, b1_ref, w2_ref, b2_ref, out_ref):
    k = pl.program_id(1)
    x = x_ref[...].astype(jnp.bfloat16)
    w1 = w1_ref[0].astype(jnp.bfloat16)
    h = jnp.dot(x, w1, preferred_element_type=jnp.float32) + b1_ref[0]
    h = jax.nn.gelu(h).astype(jnp.bfloat16)
    part = jnp.dot(h, w2_ref[0].astype(jnp.bfloat16),
                   preferred_element_type=jnp.float32)

    @pl.when(k == 0)
    def _():
        out_ref[...] = part + b2_ref[0]

    @pl.when(k != 0)
    def _():
        out_ref[...] = out_ref[...] + part


def _grouped_ffn(sorted_x, bid, eid, W1, b1, W2, b2):
    grid_spec = pltpu.PrefetchScalarGridSpec(
        num_scalar_prefetch=2,
        grid=(_NT, _NK),
        in_specs=[
            pl.BlockSpec((_BM, _D), lambda t, k, bid, eid: (bid[t], 0)),
            pl.BlockSpec((1, _D, _BK), lambda t, k, bid, eid: (eid[t], 0, _serp(t, k))),
            pl.BlockSpec((1, 1, _BK), lambda t, k, bid, eid: (eid[t], 0, _serp(t, k))),
            pl.BlockSpec((1, _BK, _D), lambda t, k, bid, eid: (eid[t], _serp(t, k), 0)),
            pl.BlockSpec((1, 1, _D), lambda t, k, bid, eid: (eid[t], 0, 0)),
        ],
        out_specs=pl.BlockSpec((_BM, _D), lambda t, k, bid, eid: (bid[t], 0)),
    )
    return pl.pallas_call(
        _ffn_body,
        grid_spec=grid_spec,
        out_shape=jax.ShapeDtypeStruct((_RP, _D), jnp.float32),
        compiler_params=pltpu.CompilerParams(
            dimension_semantics=("arbitrary", "arbitrary"),
        ),
    )(bid, eid, sorted_x, W1, b1.reshape(_E, 1, _DFF), W2, b2.reshape(_E, 1, _D))


def kernel(x, gate_W, gate_b, W1, b1, W2, b2):
    x_flat = x.reshape(_T, _D)

    logits = x_flat @ gate_W + gate_b
    top_vals, top_idx = jax.lax.top_k(logits, _K)
    w = jax.nn.softmax(top_vals.astype(jnp.float32), axis=1)

    flat_e = top_idx.reshape(-1).astype(jnp.int32)
    oh = (flat_e[:, None] == jnp.arange(_E, dtype=jnp.int32)[None, :])
    cum = jnp.cumsum(oh.astype(jnp.int32), axis=0)
    g = cum[-1]
    rank = jnp.take_along_axis(cum, flat_e[:, None], axis=1)[:, 0] - 1

    nb_e = (g + _BM - 1) // _BM
    csnb = jnp.cumsum(nb_e)
    nbp = csnb[-1]
    off_pad = jnp.concatenate([jnp.zeros((1,), jnp.int32),
                               (csnb[:-1] * _BM).astype(jnp.int32)])
    pos = off_pad[flat_e] + rank

    t_ar = jnp.arange(_NT, dtype=jnp.int32)
    bid = jnp.minimum(t_ar, nbp - 1).astype(jnp.int32)
    eid = jnp.searchsorted(csnb, bid, side="right").astype(jnp.int32)

    pos2 = pos.reshape(_T, _K)
    pos0 = pos2[:, 0]
    pos1 = pos2[:, 1]

    sorted_x = _sc_dispatch(x_flat, pos0, pos1)

    contrib = _grouped_ffn(sorted_x, bid, eid, W1, b1, W2, b2)

    w0x = jnp.broadcast_to(w[:, 0:1], (_T, 16))
    w1x = jnp.broadcast_to(w[:, 1:2], (_T, 16))
    out = _sc_combine(contrib, pos0, pos1, w0x, w1x)
    return out.reshape(_B, _S, _D)
